# baseline (speedup 1.0000x reference)
import functools

import jax
import jax.numpy as jnp
from jax import lax
from jax.experimental import pallas as pl
from jax.experimental.pallas import tpu as pltpu
from jax.experimental.pallas import tpu_sc as plsc

_TOP_K = 8
_N_ROUTED = 64
_N_GROUP = 8
_TOPK_GROUP = 4
_SCALING = 2.5
_GROUP_SIZE = _N_ROUTED // _N_GROUP

_NEG_INF = float("-inf")


def _tree_max(vals):
    vals = list(vals)
    while len(vals) > 1:
        nxt = [jnp.maximum(vals[i], vals[i + 1])
               for i in range(0, len(vals) - 1, 2)]
        if len(vals) % 2:
            nxt.append(vals[-1])
        vals = nxt
    return vals[0]


def _tree_min(vals):
    vals = list(vals)
    while len(vals) > 1:
        nxt = [jnp.minimum(vals[i], vals[i + 1])
               for i in range(0, len(vals) - 1, 2)]
        if len(vals) % 2:
            nxt.append(vals[-1])
        vals = nxt
    return vals[0]


def _route_vecs(s, shape):
    neg = jnp.full(shape, _NEG_INF, jnp.float32)

    gval = []
    for g in range(_N_GROUP):
        m1 = s[g * _GROUP_SIZE]
        m2 = neg
        for j in range(1, _GROUP_SIZE):
            x = s[g * _GROUP_SIZE + j]
            lo = jnp.minimum(m1, x)
            m1 = jnp.maximum(m1, x)
            m2 = jnp.maximum(m2, lo)
        gval.append(m1 + m2)

    sel = [None] * _N_GROUP
    gw = list(gval)
    big_g = jnp.full(shape, _N_GROUP, jnp.int32)
    for r in range(_TOPK_GROUP):
        m = _tree_max(gw)
        widx = _tree_min(
            [jnp.where(gw[g] == m, g, big_g) for g in range(_N_GROUP)])
        for g in range(_N_GROUP):
            hit = widx == g
            sel[g] = hit if r == 0 else (sel[g] | hit)
            gw[g] = jnp.where(hit, neg, gw[g])

    ms = [jnp.where(sel[e // _GROUP_SIZE], s[e], 0.0)
          for e in range(_N_ROUTED)]

    wsum = jnp.zeros(shape, jnp.float32)
    big_e = jnp.full(shape, _N_ROUTED, jnp.int32)
    idxs = []
    wts = []
    for _ in range(_TOP_K):
        m = _tree_max(ms)
        widx = _tree_min(
            [jnp.where(ms[e] == m, e, big_e) for e in range(_N_ROUTED)])
        for e in range(_N_ROUTED):
            ms[e] = jnp.where(widx == e, neg, ms[e])
        idxs.append(widx)
        wts.append(m)
        wsum = wsum + m

    inv = _SCALING / (wsum + 1e-20)
    wts = [w * inv for w in wts]
    return idxs, wts


def _score_kernel(hs_ref, wt_ref, b_ref, sc_ref):
    wt = wt_ref[...]
    b = b_ref[...]
    hs = hs_ref[...]
    logits_t = lax.dot_general(
        wt, hs, (((1,), (1,)), ((), ())),
        preferred_element_type=jnp.float32)
    sc_ref[0] = jax.nn.sigmoid(logits_t) + b


def _make_sc_router(n_tok):
    info = plsc.get_sparse_core_info()
    nc, ns, lanes = info.num_cores, info.num_subcores, info.num_lanes
    nw = nc * ns
    tok_w = n_tok // nw
    mesh = plsc.VectorSubcoreMesh(core_axis_name="c", subcore_axis_name="s")

    @functools.partial(
        pl.kernel, mesh=mesh,
        out_type=[
            jax.ShapeDtypeStruct((nw, _TOP_K, tok_w), jnp.int32),
            jax.ShapeDtypeStruct((nw, _TOP_K, tok_w), jnp.float32),
        ],
        scratch_types=[
            pltpu.VMEM((_N_ROUTED, tok_w), jnp.float32),
            pltpu.VMEM((_TOP_K, tok_w), jnp.int32),
            pltpu.VMEM((_TOP_K, tok_w), jnp.float32),
        ],
    )
    def router(sc_hbm, idx_hbm, wgt_hbm, slab, idxb, wgtb):
        wid = lax.axis_index("s") * nc + lax.axis_index("c")
        pltpu.sync_copy(sc_hbm.at[wid], slab)

        def body(t, carry):
            t16 = t * lanes
            s = [slab[e, pl.ds(t16, lanes)] for e in range(_N_ROUTED)]
            idxs, wts = _route_vecs(s, (lanes,))
            for k in range(_TOP_K):
                idxb[k, pl.ds(t16, lanes)] = idxs[k]
                wgtb[k, pl.ds(t16, lanes)] = wts[k]
            return carry

        lax.fori_loop(0, tok_w // lanes, body, 0)
        pltpu.sync_copy(idxb, idx_hbm.at[wid])
        pltpu.sync_copy(wgtb, wgt_hbm.at[wid])

    return router, nw, tok_w


def kernel(hidden_states, kernel, e_score_correction_bias):
    bsz, seq_len, h = hidden_states.shape
    n = bsz * seq_len
    hs = hidden_states.reshape(n, h)
    wt = kernel.astype(jnp.float32).T
    b2d = e_score_correction_bias.reshape(_N_ROUTED, 1).astype(jnp.float32)

    chunk = 1024
    nchunk = n // chunk
    scores = pl.pallas_call(
        _score_kernel,
        grid=(nchunk,),
        in_specs=[
            pl.BlockSpec((chunk, h), lambda i: (i, 0)),
            pl.BlockSpec((_N_ROUTED, h), lambda i: (0, 0)),
            pl.BlockSpec((_N_ROUTED, 1), lambda i: (0, 0)),
        ],
        out_specs=pl.BlockSpec((1, _N_ROUTED, chunk), lambda i: (i, 0, 0)),
        out_shape=jax.ShapeDtypeStruct((nchunk, _N_ROUTED, chunk),
                                       jnp.float32),
    )(hs, wt, b2d)

    router, nw, tok_w = _make_sc_router(n)
    assert nw == nchunk and tok_w == chunk
    idx3, wgt3 = router(scores)
    topk_idx = jnp.transpose(idx3, (0, 2, 1)).reshape(n, _TOP_K)
    topk_wgt = jnp.transpose(wgt3, (0, 2, 1)).reshape(n, _TOP_K)
    return (topk_idx, topk_wgt)


# speedup vs baseline: 2.1069x; 2.1069x over previous
import jax
import jax.numpy as jnp
from jax import lax
from jax.experimental import pallas as pl

_TOP_K = 8
_N_ROUTED = 64
_N_GROUP = 8
_TOPK_GROUP = 4
_SCALING = 2.5
_GROUP_SIZE = _N_ROUTED // _N_GROUP

_NEG_INF = float("-inf")


def _tree_max(vals):
    vals = list(vals)
    while len(vals) > 1:
        nxt = [jnp.maximum(vals[i], vals[i + 1])
               for i in range(0, len(vals) - 1, 2)]
        if len(vals) % 2:
            nxt.append(vals[-1])
        vals = nxt
    return vals[0]


def _tree_min(vals):
    vals = list(vals)
    while len(vals) > 1:
        nxt = [jnp.minimum(vals[i], vals[i + 1])
               for i in range(0, len(vals) - 1, 2)]
        if len(vals) % 2:
            nxt.append(vals[-1])
        vals = nxt
    return vals[0]


def _route_kernel(hs_ref, wt_ref, b_ref, idx_ref, wgt_ref):
    wt = wt_ref[...]
    b = b_ref[...]
    t = hs_ref.shape[0]
    chunk = 1024
    for c in range(t // chunk):
        hs = hs_ref[pl.ds(c * chunk, chunk), :]
        logits_t = lax.dot_general(
            wt, hs, (((1,), (1,)), ((), ())),
            preferred_element_type=jnp.float32)
        scores_t = jax.nn.sigmoid(logits_t) + b
        _route_chunk(scores_t, c, idx_ref, wgt_ref)


def _route_chunk(scores_t, c, idx_ref, wgt_ref):
    sub = scores_t.shape[1] // 128
    s3 = scores_t.reshape(_N_ROUTED, sub, 128)
    s = [s3[e] for e in range(_N_ROUTED)]

    shape = (sub, 128)
    neg = jnp.full(shape, _NEG_INF, jnp.float32)

    gval = []
    for g in range(_N_GROUP):
        m1 = s[g * _GROUP_SIZE]
        m2 = neg
        for j in range(1, _GROUP_SIZE):
            x = s[g * _GROUP_SIZE + j]
            lo = jnp.minimum(m1, x)
            m1 = jnp.maximum(m1, x)
            m2 = jnp.maximum(m2, lo)
        gval.append(m1 + m2)

    sel = [None] * _N_GROUP
    gw = list(gval)
    big_g = jnp.full(shape, _N_GROUP, jnp.int32)
    for r in range(_TOPK_GROUP):
        m = _tree_max(gw)
        widx = _tree_min(
            [jnp.where(gw[g] == m, g, big_g) for g in range(_N_GROUP)])
        for g in range(_N_GROUP):
            hit = widx == g
            sel[g] = hit if r == 0 else (sel[g] | hit)
            gw[g] = jnp.where(hit, neg, gw[g])

    ms = [jnp.where(sel[e // _GROUP_SIZE], s[e], 0.0)
          for e in range(_N_ROUTED)]

    wsum = jnp.zeros(shape, jnp.float32)
    big_e = jnp.full(shape, _N_ROUTED, jnp.int32)
    idxs = []
    wts = []
    for _ in range(_TOP_K):
        m = _tree_max(ms)
        widx = _tree_min(
            [jnp.where(ms[e] == m, e, big_e) for e in range(_N_ROUTED)])
        for e in range(_N_ROUTED):
            ms[e] = jnp.where(widx == e, neg, ms[e])
        idxs.append(widx)
        wts.append(m)
        wsum = wsum + m

    inv = _SCALING / (wsum + 1e-20)
    for k in range(_TOP_K):
        idx_ref[k, c] = idxs[k]
        wgt_ref[k, c] = wts[k] * inv


def kernel(hidden_states, kernel, e_score_correction_bias):
    bsz, seq_len, h = hidden_states.shape
    n = bsz * seq_len
    hs = hidden_states.reshape(n, h)
    wt = kernel.astype(jnp.float32).T
    b2d = e_score_correction_bias.reshape(_N_ROUTED, 1).astype(jnp.float32)

    block_t = 4096
    chunks_per_blk = block_t // 1024
    sub = 1024 // 128
    nchunk = n // 1024
    grid = (n // block_t,)
    out_shape = [
        jax.ShapeDtypeStruct((_TOP_K, nchunk, sub, 128), jnp.int32),
        jax.ShapeDtypeStruct((_TOP_K, nchunk, sub, 128), jnp.float32),
    ]
    idx4, wgt4 = pl.pallas_call(
        _route_kernel,
        grid=grid,
        in_specs=[
            pl.BlockSpec((block_t, h), lambda i: (i, 0)),
            pl.BlockSpec((_N_ROUTED, h), lambda i: (0, 0)),
            pl.BlockSpec((_N_ROUTED, 1), lambda i: (0, 0)),
        ],
        out_specs=[
            pl.BlockSpec((_TOP_K, chunks_per_blk, sub, 128),
                         lambda i: (0, i, 0, 0)),
            pl.BlockSpec((_TOP_K, chunks_per_blk, sub, 128),
                         lambda i: (0, i, 0, 0)),
        ],
        out_shape=out_shape,
    )(hs, wt, b2d)
    topk_idx = jnp.transpose(idx4, (1, 2, 3, 0)).reshape(n, _TOP_K)
    topk_wgt = jnp.transpose(wgt4, (1, 2, 3, 0)).reshape(n, _TOP_K)
    return (topk_idx, topk_wgt)


# speedup vs baseline: 2.1210x; 1.0067x over previous
import jax
import jax.numpy as jnp
from jax import lax
from jax.experimental import pallas as pl

_TOP_K = 8
_N_ROUTED = 64
_N_GROUP = 8
_TOPK_GROUP = 4
_SCALING = 2.5
_GROUP_SIZE = _N_ROUTED // _N_GROUP

_NEG_INF = float("-inf")


def _tree_max(vals):
    vals = list(vals)
    while len(vals) > 1:
        nxt = [jnp.maximum(vals[i], vals[i + 1])
               for i in range(0, len(vals) - 1, 2)]
        if len(vals) % 2:
            nxt.append(vals[-1])
        vals = nxt
    return vals[0]


def _tree_min(vals):
    vals = list(vals)
    while len(vals) > 1:
        nxt = [jnp.minimum(vals[i], vals[i + 1])
               for i in range(0, len(vals) - 1, 2)]
        if len(vals) % 2:
            nxt.append(vals[-1])
        vals = nxt
    return vals[0]


def _route_kernel(hs_ref, wt_ref, b_ref, idx_ref, wgt_ref):
    wt = wt_ref[...]
    b = b_ref[...]
    t = hs_ref.shape[0]
    chunk = 2048
    for c in range(t // chunk):
        hs = hs_ref[pl.ds(c * chunk, chunk), :]
        logits_t = lax.dot_general(
            wt, hs, (((1,), (1,)), ((), ())),
            preferred_element_type=jnp.float32)
        scores_t = jax.nn.sigmoid(logits_t) + b
        _route_chunk(scores_t, c, idx_ref, wgt_ref)


def _route_chunk(scores_t, c, idx_ref, wgt_ref):
    sub = scores_t.shape[1] // 128
    s3 = scores_t.reshape(_N_ROUTED, sub, 128)
    s = [s3[e] for e in range(_N_ROUTED)]

    shape = (sub, 128)
    neg = jnp.full(shape, _NEG_INF, jnp.float32)

    gval = []
    for g in range(_N_GROUP):
        m1 = s[g * _GROUP_SIZE]
        m2 = neg
        for j in range(1, _GROUP_SIZE):
            x = s[g * _GROUP_SIZE + j]
            lo = jnp.minimum(m1, x)
            m1 = jnp.maximum(m1, x)
            m2 = jnp.maximum(m2, lo)
        gval.append(m1 + m2)

    sel = [None] * _N_GROUP
    gw = list(gval)
    big_g = jnp.full(shape, _N_GROUP, jnp.int32)
    for r in range(_TOPK_GROUP):
        m = _tree_max(gw)
        widx = _tree_min(
            [jnp.where(gw[g] == m, g, big_g) for g in range(_N_GROUP)])
        for g in range(_N_GROUP):
            hit = widx == g
            sel[g] = hit if r == 0 else (sel[g] | hit)
            gw[g] = jnp.where(hit, neg, gw[g])

    ms = [jnp.where(sel[e // _GROUP_SIZE], s[e], 0.0)
          for e in range(_N_ROUTED)]

    wsum = jnp.zeros(shape, jnp.float32)
    big_e = jnp.full(shape, _N_ROUTED, jnp.int32)
    idxs = []
    wts = []
    for _ in range(_TOP_K):
        m = _tree_max(ms)
        widx = _tree_min(
            [jnp.where(ms[e] == m, e, big_e) for e in range(_N_ROUTED)])
        for e in range(_N_ROUTED):
            ms[e] = jnp.where(widx == e, neg, ms[e])
        idxs.append(widx)
        wts.append(m)
        wsum = wsum + m

    inv = _SCALING / (wsum + 1e-20)
    for k in range(_TOP_K):
        idx_ref[k, c] = idxs[k]
        wgt_ref[k, c] = wts[k] * inv


def kernel(hidden_states, kernel, e_score_correction_bias):
    bsz, seq_len, h = hidden_states.shape
    n = bsz * seq_len
    hs = hidden_states.reshape(n, h)
    wt = kernel.astype(jnp.float32).T
    b2d = e_score_correction_bias.reshape(_N_ROUTED, 1).astype(jnp.float32)

    block_t = 4096
    chunks_per_blk = block_t // 2048
    sub = 2048 // 128
    nchunk = n // 2048
    grid = (n // block_t,)
    out_shape = [
        jax.ShapeDtypeStruct((_TOP_K, nchunk, sub, 128), jnp.int32),
        jax.ShapeDtypeStruct((_TOP_K, nchunk, sub, 128), jnp.float32),
    ]
    idx4, wgt4 = pl.pallas_call(
        _route_kernel,
        grid=grid,
        in_specs=[
            pl.BlockSpec((block_t, h), lambda i: (i, 0)),
            pl.BlockSpec((_N_ROUTED, h), lambda i: (0, 0)),
            pl.BlockSpec((_N_ROUTED, 1), lambda i: (0, 0)),
        ],
        out_specs=[
            pl.BlockSpec((_TOP_K, chunks_per_blk, sub, 128),
                         lambda i: (0, i, 0, 0)),
            pl.BlockSpec((_TOP_K, chunks_per_blk, sub, 128),
                         lambda i: (0, i, 0, 0)),
        ],
        out_shape=out_shape,
    )(hs, wt, b2d)
    topk_idx = jnp.transpose(idx4, (1, 2, 3, 0)).reshape(n, _TOP_K)
    topk_wgt = jnp.transpose(wgt4, (1, 2, 3, 0)).reshape(n, _TOP_K)
    return (topk_idx, topk_wgt)
